# baseline (speedup 1.0000x reference)
import functools

import jax
import jax.numpy as jnp
from jax import lax
from jax.experimental import pallas as pl
from jax.experimental.pallas import tpu as pltpu
from jax.experimental.pallas import tpu_sc as plsc

B = 16384
D = 64
NROWS = 1_000_000
WPR = D // 2
NC = 2
NS = 16
L = 16
NW = NC * NS
BPW = B // NW
CHUNK = 128
NCHUNK = BPW // CHUNK
GROUPS = BPW // L


def _mf_kernel(user_hbm, item_hbm, upk_hbm, ipk_hbm, out_hbm,
               uidx_v, iidx_v, urows_v, irows_v, out_v, sem):
    wid = lax.axis_index("s") * NC + lax.axis_index("c")
    base = wid * BPW

    for j in range(NCHUNK):
        pltpu.sync_copy(user_hbm.at[pl.ds(base + j * CHUNK, CHUNK)],
                        uidx_v.at[j])
        pltpu.sync_copy(item_hbm.at[pl.ds(base + j * CHUNK, CHUNK)],
                        iidx_v.at[j])

    copies = []
    for j in range(NCHUNK):
        copies.append(pltpu.async_copy(
            upk_hbm.at[uidx_v.at[j]],
            urows_v.at[pl.ds(j * CHUNK, CHUNK)], sem))
        copies.append(pltpu.async_copy(
            ipk_hbm.at[iidx_v.at[j]],
            irows_v.at[pl.ds(j * CHUNK, CHUNK)], sem))
    for c in copies:
        c.wait()

    iota = lax.iota(jnp.int32, L)

    def group(g, carry):
        ridx = g * L + iota
        acc_e = jnp.zeros((L,), jnp.float32)
        acc_o = jnp.zeros((L,), jnp.float32)
        for w in range(WPR):
            cidx = jnp.bitwise_and(iota + w, WPR - 1)
            u = plsc.load_gather(urows_v, [ridx, cidx])
            it = plsc.load_gather(irows_v, [ridx, cidx])
            ue = plsc.bitcast(u << 16, jnp.float32)
            uo = plsc.bitcast(jnp.bitwise_and(u, -65536), jnp.float32)
            ie = plsc.bitcast(it << 16, jnp.float32)
            io = plsc.bitcast(jnp.bitwise_and(it, -65536), jnp.float32)
            acc_e = acc_e + ue * ie
            acc_o = acc_o + uo * io
        out_v[pl.ds(g * L, L)] = acc_e + acc_o
        return carry

    lax.fori_loop(0, GROUPS, group, 0)

    pltpu.sync_copy(out_v, out_hbm.at[pl.ds(base, BPW)])


@jax.jit
def kernel(user, item, user_emb, item_emb):
    def repack(t):
        tb = t.astype(jnp.bfloat16).reshape(NROWS, WPR, 2)
        return jax.lax.bitcast_convert_type(tb, jnp.int32)

    f = pl.kernel(
        _mf_kernel,
        out_type=jax.ShapeDtypeStruct((B,), jnp.float32),
        mesh=plsc.VectorSubcoreMesh(core_axis_name="c", subcore_axis_name="s"),
        compiler_params=pltpu.CompilerParams(
            use_tc_tiling_on_sc=False, needs_layout_passes=False),
        scratch_types=[
            pltpu.VMEM((NCHUNK, CHUNK), jnp.int32),
            pltpu.VMEM((NCHUNK, CHUNK), jnp.int32),
            pltpu.VMEM((BPW, WPR), jnp.int32),
            pltpu.VMEM((BPW, WPR), jnp.int32),
            pltpu.VMEM((BPW,), jnp.float32),
            pltpu.SemaphoreType.DMA,
        ],
    )
    return f(user.astype(jnp.int32), item.astype(jnp.int32),
             repack(user_emb), repack(item_emb))
